# baseline (speedup 1.0000x reference)
import functools

import jax
import jax.numpy as jnp
from jax import lax
from jax.experimental import pallas as pl
from jax.experimental.pallas import tpu as pltpu
from jax.experimental.pallas import tpu_sc as plsc

N = 4096
D = 16
BQ = 1024
GRID = N // BQ
INF = float("inf")


BK = 1024
GJ = N // BK


def _nn_idx_body(q_ref, x_ref, idx_ref, x2_ref, m_ref, ix_ref):
    i = pl.program_id(0)
    j = pl.program_id(1)

    @pl.when(j == 0)
    def _init():
        m_ref[...] = jnp.full((BQ, 1), INF, jnp.float32)
        ix_ref[...] = jnp.full((BQ, 1), N, jnp.int32)

    xb = x_ref[...]

    @pl.when(i == 0)
    def _build_x2():
        x2 = jnp.sum(xb * xb, axis=1, keepdims=True)
        x2_ref[0, pl.ds(j * BK, BK)] = x2.reshape(BK)

    q = q_ref[...]
    g = lax.dot_general(q, xb, (((1,), (1,)), ((), ())),
                        preferred_element_type=jnp.float32)
    q2 = jnp.sum(q * q, axis=1, keepdims=True)
    x2l = x2_ref[0, pl.ds(j * BK, BK)].reshape(1, BK)
    d2 = q2 + x2l - 2.0 * g

    def _mask_diag(t):
        cols = lax.broadcasted_iota(jnp.int32, (BQ, BK), 1)
        rows = lax.broadcasted_iota(jnp.int32, (BQ, BK), 0)
        return jnp.where(cols == rows, INF, t)

    d2 = lax.cond(i == j, _mask_diag, lambda t: t, d2)

    mb = jnp.min(d2, axis=1, keepdims=True)
    cols = j * BK + lax.broadcasted_iota(jnp.int32, (BQ, BK), 1)
    ib = jnp.min(jnp.where(d2 <= mb, cols, 2 * N), axis=1,
                 keepdims=True)
    better = mb < m_ref[...]
    ix_ref[...] = jnp.where(better, ib, ix_ref[...])
    m_ref[...] = jnp.where(better, mb, m_ref[...])

    @pl.when(j == GJ - 1)
    def _flush():
        idx_ref[...] = ix_ref[...].reshape(1, 1, BQ)


def _nn_indices(x):
    return pl.pallas_call(
        _nn_idx_body,
        grid=(GRID, GJ),
        in_specs=[
            pl.BlockSpec((BQ, D), lambda i, j: (i, 0)),
            pl.BlockSpec((BK, D), lambda i, j: (j, 0)),
        ],
        out_specs=pl.BlockSpec((1, 1, BQ), lambda i, j: (i, 0, 0)),
        out_shape=jax.ShapeDtypeStruct((GRID, 1, BQ), jnp.int32),
        scratch_shapes=[
            pltpu.VMEM((1, N), jnp.float32),
            pltpu.VMEM((BQ, 1), jnp.float32),
            pltpu.VMEM((BQ, 1), jnp.int32),
        ],
    )(x, x)


def _make_sc_gather():
    info = plsc.get_sparse_core_info()
    nw = info.num_cores * info.num_subcores
    b_per_w = N // nw
    wpg = BQ // b_per_w
    mesh = plsc.VectorSubcoreMesh(core_axis_name="c", subcore_axis_name="s")

    @functools.partial(
        pl.kernel,
        mesh=mesh,
        compiler_params=pltpu.CompilerParams(use_tc_tiling_on_sc=False),
        out_type=jax.ShapeDtypeStruct((N, D), jnp.float32),
        scratch_types=[
            pltpu.VMEM((b_per_w,), jnp.int32),
            pltpu.VMEM((b_per_w, D), jnp.float32),
            pltpu.SemaphoreType.DMA,
        ],
    )
    def gather(table_hbm, idx_hbm, out_hbm, idx_v, rows_v, sem):
        wid = lax.axis_index("s") * info.num_cores + lax.axis_index("c")
        g = wid // wpg
        off = (wid % wpg) * b_per_w
        pltpu.sync_copy(idx_hbm.at[g, 0, pl.ds(off, b_per_w)], idx_v)
        pltpu.async_copy(table_hbm.at[idx_v], rows_v, sem).wait()
        pltpu.sync_copy(rows_v, out_hbm.at[pl.ds(wid * b_per_w, b_per_w)])

    return gather


_sc_gather = None


def kernel(x, queue_buf):
    del queue_buf
    global _sc_gather
    if _sc_gather is None:
        _sc_gather = _make_sc_gather()
    idx3 = _nn_indices(x)
    return _sc_gather(x, idx3)


# speedup vs baseline: 1.0014x; 1.0014x over previous
import functools

import jax
import jax.numpy as jnp
from jax import lax
from jax.experimental import pallas as pl
from jax.experimental.pallas import tpu as pltpu
from jax.experimental.pallas import tpu_sc as plsc

N = 4096
D = 16
BQ = 1024
GRID = N // BQ
INF = float("inf")


def _nn_idx_body(q_ref, x_ref, idx_ref, x2_ref):
    i = pl.program_id(0)

    @pl.when(i == 0)
    def _build_x2():
        xf = x_ref[...]
        x2 = jnp.sum(xf * xf, axis=1, keepdims=True)
        x2_ref[...] = x2.reshape(1, N)

    q = q_ref[...]
    g = lax.dot_general(q, x_ref[...], (((1,), (1,)), ((), ())),
                        preferred_element_type=jnp.float32)
    q2 = jnp.sum(q * q, axis=1, keepdims=True)
    x2l = x2_ref[...]
    d2 = q2 + x2l - 2.0 * g

    def _mask_win(k):
        def f(t):
            sub = t[:, k * BQ:(k + 1) * BQ]
            c = lax.broadcasted_iota(jnp.int32, (BQ, BQ), 1)
            r = lax.broadcasted_iota(jnp.int32, (BQ, BQ), 0)
            sub = jnp.where(c == r, INF, sub)
            parts = []
            if k > 0:
                parts.append(t[:, :k * BQ])
            parts.append(sub)
            if k < GRID - 1:
                parts.append(t[:, (k + 1) * BQ:])
            return jnp.concatenate(parts, axis=1)
        return f

    d2 = lax.switch(i, [_mask_win(k) for k in range(GRID)], d2)

    m = jnp.min(d2, axis=1, keepdims=True)
    idx = jnp.min(jnp.where(d2 <= m, cols_iota(), 2 * N), axis=1)
    idx_ref[...] = idx.astype(jnp.int32).reshape(1, 1, BQ)


def cols_iota():
    return lax.broadcasted_iota(jnp.int32, (BQ, N), 1)


def _nn_indices(x):
    return pl.pallas_call(
        _nn_idx_body,
        grid=(GRID,),
        in_specs=[
            pl.BlockSpec((BQ, D), lambda i: (i, 0)),
            pl.BlockSpec((N, D), lambda i: (0, 0)),
        ],
        out_specs=pl.BlockSpec((1, 1, BQ), lambda i: (i, 0, 0)),
        out_shape=jax.ShapeDtypeStruct((GRID, 1, BQ), jnp.int32),
        scratch_shapes=[pltpu.VMEM((1, N), jnp.float32)],
    )(x, x)


def _make_sc_gather():
    info = plsc.get_sparse_core_info()
    nw = info.num_cores * info.num_subcores
    b_per_w = N // nw
    wpg = BQ // b_per_w
    mesh = plsc.VectorSubcoreMesh(core_axis_name="c", subcore_axis_name="s")

    @functools.partial(
        pl.kernel,
        mesh=mesh,
        compiler_params=pltpu.CompilerParams(use_tc_tiling_on_sc=False),
        out_type=jax.ShapeDtypeStruct((N, D), jnp.float32),
        scratch_types=[
            pltpu.VMEM((b_per_w,), jnp.int32),
            pltpu.VMEM((b_per_w, D), jnp.float32),
            pltpu.SemaphoreType.DMA,
        ],
    )
    def gather(table_hbm, idx_hbm, out_hbm, idx_v, rows_v, sem):
        wid = lax.axis_index("s") * info.num_cores + lax.axis_index("c")
        g = wid // wpg
        off = (wid % wpg) * b_per_w
        pltpu.sync_copy(idx_hbm.at[g, 0, pl.ds(off, b_per_w)], idx_v)
        pltpu.async_copy(table_hbm.at[idx_v], rows_v, sem).wait()
        pltpu.sync_copy(rows_v, out_hbm.at[pl.ds(wid * b_per_w, b_per_w)])

    return gather


_sc_gather = None


def kernel(x, queue_buf):
    del queue_buf
    global _sc_gather
    if _sc_gather is None:
        _sc_gather = _make_sc_gather()
    idx3 = _nn_indices(x)
    return _sc_gather(x, idx3)


# speedup vs baseline: 1.3818x; 1.3798x over previous
import functools

import jax
import jax.numpy as jnp
from jax import lax
from jax.experimental import pallas as pl
from jax.experimental.pallas import tpu as pltpu
from jax.experimental.pallas import tpu_sc as plsc

N = 4096
D = 16
BQ = 1024
GRID = N // BQ
INF = float("inf")


def _nn_idx_body(q_ref, x_ref, idx_ref, x2_ref):
    i = pl.program_id(0)

    @pl.when(i == 0)
    def _build_x2():
        xf = x_ref[...]
        x2 = jnp.sum(xf * xf, axis=1, keepdims=True)
        x2_ref[...] = x2.reshape(1, N)

    q = q_ref[...]
    g = lax.dot_general(q, x_ref[...], (((1,), (1,)), ((), ())),
                        preferred_element_type=jnp.float32)
    q2 = jnp.sum(q * q, axis=1, keepdims=True)
    G8 = BQ // 8
    s1 = (q2.reshape(G8, 8, 1) + x2_ref[...].reshape(1, 1, N))
    d2 = (s1 - 2.0 * g.reshape(G8, 8, N)).reshape(BQ, N)

    cols = lax.broadcasted_iota(jnp.int32, (BQ, N), 1)
    rows = i * BQ + lax.broadcasted_iota(jnp.int32, (BQ, N), 0)
    d2 = jnp.where(cols == rows, INF, d2)

    m = jnp.min(d2, axis=1, keepdims=True)
    idx = jnp.min(jnp.where(d2 <= m, cols, 2 * N), axis=1)
    idx_ref[...] = idx.reshape(1, 1, BQ)


def _nn_indices(x):
    return pl.pallas_call(
        _nn_idx_body,
        grid=(GRID,),
        in_specs=[
            pl.BlockSpec((BQ, D), lambda i: (i, 0)),
            pl.BlockSpec((N, D), lambda i: (0, 0)),
        ],
        out_specs=pl.BlockSpec((1, 1, BQ), lambda i: (i, 0, 0)),
        out_shape=jax.ShapeDtypeStruct((GRID, 1, BQ), jnp.int32),
        scratch_shapes=[pltpu.VMEM((1, N), jnp.float32)],
    )(x, x)


def _make_sc_gather():
    info = plsc.get_sparse_core_info()
    nw = info.num_cores * info.num_subcores
    b_per_w = N // nw
    wpg = BQ // b_per_w
    mesh = plsc.VectorSubcoreMesh(core_axis_name="c", subcore_axis_name="s")

    @functools.partial(
        pl.kernel,
        mesh=mesh,
        compiler_params=pltpu.CompilerParams(use_tc_tiling_on_sc=False),
        out_type=jax.ShapeDtypeStruct((N, D), jnp.float32),
        scratch_types=[
            pltpu.VMEM((b_per_w,), jnp.int32),
            pltpu.VMEM((b_per_w, D), jnp.float32),
            pltpu.SemaphoreType.DMA,
        ],
    )
    def gather(table_hbm, idx_hbm, out_hbm, idx_v, rows_v, sem):
        wid = lax.axis_index("s") * info.num_cores + lax.axis_index("c")
        g = wid // wpg
        off = (wid % wpg) * b_per_w
        pltpu.sync_copy(idx_hbm.at[g, 0, pl.ds(off, b_per_w)], idx_v)
        pltpu.async_copy(table_hbm.at[idx_v], rows_v, sem).wait()
        pltpu.sync_copy(rows_v, out_hbm.at[pl.ds(wid * b_per_w, b_per_w)])

    return gather


_sc_gather = None


def kernel(x, queue_buf):
    del queue_buf
    global _sc_gather
    if _sc_gather is None:
        _sc_gather = _make_sc_gather()
    idx3 = _nn_indices(x)
    return _sc_gather(x, idx3)


# speedup vs baseline: 1.4551x; 1.0530x over previous
import functools

import jax
import jax.numpy as jnp
from jax import lax
from jax.experimental import pallas as pl
from jax.experimental.pallas import tpu as pltpu
from jax.experimental.pallas import tpu_sc as plsc

N = 4096
D = 16
BQ = 1024
GRID = N // BQ
INF = float("inf")


def _nn_idx_body(q_ref, x_ref, idx_ref, x2_ref):
    i = pl.program_id(0)

    @pl.when(i == 0)
    def _build_x2():
        xf = x_ref[...]
        x2 = jnp.sum(xf * xf, axis=1, keepdims=True)
        x2_ref[...] = x2.reshape(1, N)

    q = q_ref[...]
    g = lax.dot_general(q, x_ref[...], (((1,), (1,)), ((), ())),
                        preferred_element_type=jnp.float32)
    q2 = jnp.sum(q * q, axis=1, keepdims=True)
    G8 = BQ // 8
    s1 = (q2.reshape(G8, 8, 1) + x2_ref[...].reshape(1, 1, N))
    d2 = (s1 - 2.0 * g.reshape(G8, 8, N)).reshape(BQ, N)

    cols = lax.broadcasted_iota(jnp.int32, (BQ, N), 1)
    rows = i * BQ + lax.broadcasted_iota(jnp.int32, (BQ, N), 0)
    d2 = jnp.where(cols == rows, INF, d2)

    m = jnp.min(d2, axis=1, keepdims=True)
    idx = jnp.min(jnp.where(d2 <= m, cols, 2 * N), axis=1)
    idx_ref[...] = idx.reshape(BQ // 128, 128)


def _nn_indices(x):
    return pl.pallas_call(
        _nn_idx_body,
        grid=(GRID,),
        in_specs=[
            pl.BlockSpec((BQ, D), lambda i: (i, 0)),
            pl.BlockSpec((N, D), lambda i: (0, 0)),
        ],
        out_specs=pl.BlockSpec((BQ // 128, 128), lambda i: (i, 0)),
        out_shape=jax.ShapeDtypeStruct((N // 128, 128), jnp.int32),
        scratch_shapes=[pltpu.VMEM((1, N), jnp.float32)],
    )(x, x)


def _make_sc_gather():
    info = plsc.get_sparse_core_info()
    nw = info.num_cores * info.num_subcores
    b_per_w = N // nw
    wpg = BQ // b_per_w
    mesh = plsc.VectorSubcoreMesh(core_axis_name="c", subcore_axis_name="s")

    @functools.partial(
        pl.kernel,
        mesh=mesh,
        compiler_params=pltpu.CompilerParams(use_tc_tiling_on_sc=False),
        out_type=jax.ShapeDtypeStruct((N, D), jnp.float32),
        scratch_types=[
            pltpu.VMEM((b_per_w,), jnp.int32),
            pltpu.VMEM((b_per_w, D), jnp.float32),
            pltpu.SemaphoreType.DMA,
        ],
    )
    def gather(table_hbm, idx_hbm, out_hbm, idx_v, rows_v, sem):
        wid = lax.axis_index("s") * info.num_cores + lax.axis_index("c")
        pltpu.sync_copy(idx_hbm.at[wid], idx_v)
        pltpu.async_copy(table_hbm.at[idx_v], rows_v, sem).wait()
        pltpu.sync_copy(rows_v, out_hbm.at[pl.ds(wid * b_per_w, b_per_w)])

    return gather


_sc_gather = None


def kernel(x, queue_buf):
    del queue_buf
    global _sc_gather
    if _sc_gather is None:
        _sc_gather = _make_sc_gather()
    idx3 = _nn_indices(x)
    return _sc_gather(x, idx3)
